# baseline (speedup 1.0000x reference)
import functools

import jax
import jax.numpy as jnp
from jax import lax
from jax.experimental import pallas as pl
from jax.experimental.pallas import tpu as pltpu
from jax.experimental.pallas import tpu_sc as plsc

NUM_EXPERTS = 8
TOKENS = 2048
D_MODEL = 1024
D_FF = 2048
TOPK = 2

TM = 128
NT = TOKEN_TILES = (TOKENS * TOPK) // TM
SSTEPS = NT + NUM_EXPERTS - 1

NC, NS = 2, 16
NW = NC * NS


def _shift_down(a, sh):
    return jnp.concatenate([jnp.zeros((sh, a.shape[1]), a.dtype), a[:-sh, :]], axis=0)


def _lane_incl_cumsum(a):
    for sh in (1, 2, 4):
        a = a + jnp.concatenate([jnp.zeros((a.shape[0], sh), a.dtype), a[:, :-sh]], axis=1)
    return a


def _route_kernel(score_ref, pos_ref, wa_ref, wb_ref, sched_ref):
    E, S = NUM_EXPERTS, SSTEPS
    s = score_ref[...].astype(jnp.float32)
    lane = lax.broadcasted_iota(jnp.int32, s.shape, 1)
    big = jnp.asarray(E, jnp.int32)
    m1 = jnp.max(s, axis=1, keepdims=True)
    i1 = jnp.min(jnp.where(s == m1, lane, big), axis=1, keepdims=True)
    oh1 = lane == i1
    s2 = jnp.where(oh1, -jnp.inf, s)
    m2 = jnp.max(s2, axis=1, keepdims=True)
    i2 = jnp.min(jnp.where(s2 == m2, lane, big), axis=1, keepdims=True)
    oh2 = lane == i2
    r = jnp.exp(m2 - m1)
    denom = 1.0 + r
    ones16 = jnp.ones((1, 16), jnp.float32)
    wa_ref[...] = (1.0 / denom) * ones16
    wb_ref[...] = (r / denom) * ones16

    A = jnp.concatenate([oh1, oh2], axis=0).astype(jnp.float32)
    inc = A
    sh = 1
    while sh < A.shape[0]:
        inc = inc + _shift_down(inc, sh)
        sh *= 2
    counts = inc[A.shape[0] - 1:, :]
    offs_incl = _lane_incl_cumsum(counts)
    offs_excl = offs_incl - counts
    ex_rank = inc - A
    pos = jnp.sum((ex_rank + offs_excl) * A, axis=1, keepdims=True)
    pos_ref[...] = pos.astype(jnp.int32)

    off = offs_excl.astype(jnp.int32)
    offn = offs_incl.astype(jnp.int32)
    cnt = counts.astype(jnp.int32)
    nz = cnt > 0
    t0 = off // TM
    t1 = jnp.where(nz, (offn - 1) // TM, 0)
    ntiles = jnp.where(nz, t1 - t0 + 1, 0)
    cumincl = _lane_incl_cumsum(ntiles)
    cumex = cumincl - ntiles
    total = cumincl[:, E - 1:]

    sub = lax.broadcasted_iota(jnp.int32, (E, E), 0)
    lan = lax.broadcasted_iota(jnp.int32, (E, E), 1)
    ident = (sub == lan).astype(jnp.float32)

    def _t(v):
        return lax.dot_general(ident, v.astype(jnp.float32), (((1,), (1,)), ((), ())),
                               preferred_element_type=jnp.float32).astype(jnp.int32)

    cuminclT, cumexT, t0T, offT, offnT = map(_t, (cumincl, cumex, t0, off, offn))
    s_iota = lax.broadcasted_iota(jnp.int32, (1, S), 1)
    e_s = jnp.sum((s_iota >= cuminclT).astype(jnp.int32), axis=0, keepdims=True)
    e_s = jnp.minimum(e_s, E - 1)
    ohE = lax.broadcasted_iota(jnp.int32, (E, S), 0) == e_s

    def _sel(vT):
        return jnp.sum(jnp.where(ohE, vT, 0), axis=0, keepdims=True)

    tile = _sel(t0T) + s_iota - _sel(cumexT)
    valid = s_iota < total
    tile = jnp.where(valid, tile, NT - 1)
    start = jnp.where(valid, jnp.maximum(_sel(offT), tile * TM), 0)
    end = jnp.where(valid, jnp.minimum(_sel(offnT), (tile + 1) * TM), 0)
    prev_tile = jnp.concatenate([jnp.full((1, 1), -1, jnp.int32), tile[:, :-1]], axis=1)
    first = (tile != prev_tile).astype(jnp.int32)
    sched_ref[...] = jnp.concatenate([e_s, tile, start, end, first], axis=0)


def _gmm_kernel(sched_ref, x_ref, w1_ref, w2_ref, y_ref):
    s = pl.program_id(0)

    @pl.when(sched_ref[4, s] == 1)
    def _():
        y_ref[...] = jnp.zeros_like(y_ref)

    x = x_ref[...]
    gate = lax.dot_general(x, w1_ref[0, :D_FF, :], (((1,), (1,)), ((), ())),
                           preferred_element_type=jnp.float32)
    up = lax.dot_general(x, w1_ref[0, D_FF:, :], (((1,), (1,)), ((), ())),
                         preferred_element_type=jnp.float32)
    h = gate * (1.0 / (1.0 + jnp.exp(-gate))) * up
    rows = sched_ref[1, s] * TM + lax.broadcasted_iota(jnp.int32, (TM, 1), 0)
    mask = (rows >= sched_ref[2, s]) & (rows < sched_ref[3, s])
    h = jnp.where(mask, h, 0.0)
    y_ref[...] += lax.dot_general(h, w2_ref[0], (((1,), (1,)), ((), ())),
                                  preferred_element_type=jnp.float32)


def _sc_mesh():
    return plsc.VectorSubcoreMesh(core_axis_name="c", subcore_axis_name="s")


def _permute(hidden, pos1):
    @functools.partial(
        pl.kernel, mesh=_sc_mesh(),
        out_type=jax.ShapeDtypeStruct((TOKENS * TOPK, D_MODEL), jnp.float32),
        scratch_types=[
            pltpu.VMEM((64,), jnp.int32),
            pltpu.VMEM((64, D_MODEL), jnp.float32),
            pltpu.SemaphoreType.DMA,
        ],
    )
    def _permute_kernel(hid_hbm, pos_hbm, xs_hbm, idx_v, rows_v, sem):
        wid = lax.axis_index("s") * NC + lax.axis_index("c")
        for c in range(2):
            jb = wid * 128 + c * 64
            tb = lax.rem(jb, TOKENS)
            pltpu.sync_copy(hid_hbm.at[pl.ds(tb, 64)], rows_v)
            pltpu.sync_copy(pos_hbm.at[pl.ds(jb, 64)], idx_v)
            pltpu.async_copy(rows_v, xs_hbm.at[idx_v], sem).wait()

    return _permute_kernel(hidden, pos1)


def _combine(y, pa, pb, wab, wbb):
    @functools.partial(
        pl.kernel, mesh=_sc_mesh(),
        out_type=jax.ShapeDtypeStruct((TOKENS, D_MODEL), jnp.float32),
        scratch_types=[
            pltpu.VMEM((32,), jnp.int32),
            pltpu.VMEM((32,), jnp.int32),
            pltpu.VMEM((32, D_MODEL), jnp.float32),
            pltpu.VMEM((32, D_MODEL), jnp.float32),
            pltpu.VMEM((32, 16), jnp.float32),
            pltpu.VMEM((32, 16), jnp.float32),
            pltpu.SemaphoreType.DMA,
        ],
    )
    def _combine_kernel(y_hbm, pa_hbm, pb_hbm, wa_hbm, wb_hbm, out_hbm,
                        idxa_v, idxb_v, a_v, b_v, wa_v, wb_v, sem):
        wid = lax.axis_index("s") * NC + lax.axis_index("c")
        for c in range(2):
            tb = wid * 64 + c * 32
            pltpu.sync_copy(pa_hbm.at[pl.ds(tb, 32)], idxa_v)
            pltpu.sync_copy(pb_hbm.at[pl.ds(tb, 32)], idxb_v)
            pltpu.sync_copy(wa_hbm.at[pl.ds(tb, 32)], wa_v)
            pltpu.sync_copy(wb_hbm.at[pl.ds(tb, 32)], wb_v)
            ca = pltpu.async_copy(y_hbm.at[idxa_v], a_v, sem)
            cb = pltpu.async_copy(y_hbm.at[idxb_v], b_v, sem)
            ca.wait()
            cb.wait()

            def row_body(rr, carry):
                wa = wa_v[rr]
                wb = wb_v[rr]
                for cc in range(D_MODEL // 16):
                    sl = pl.ds(cc * 16, 16)
                    a_v[rr, sl] = wa * a_v[rr, sl] + wb * b_v[rr, sl]
                return carry

            lax.fori_loop(0, 32, row_body, 0)
            pltpu.sync_copy(a_v, out_hbm.at[pl.ds(tb, 32)])

    return _combine_kernel(y, pa, pb, wab, wbb)


def _route(score):
    return pl.pallas_call(
        _route_kernel,
        out_shape=(
            jax.ShapeDtypeStruct((TOKENS * TOPK, 1), jnp.int32),
            jax.ShapeDtypeStruct((TOKENS, 16), jnp.float32),
            jax.ShapeDtypeStruct((TOKENS, 16), jnp.float32),
            jax.ShapeDtypeStruct((5, SSTEPS), jnp.int32),
        ),
    )(score)


def _gmm(sched, xs, w1, w2):
    T2, D = xs.shape
    grid_spec = pltpu.PrefetchScalarGridSpec(
        num_scalar_prefetch=1,
        grid=(SSTEPS,),
        in_specs=[
            pl.BlockSpec((TM, D), lambda s, sr: (sr[1, s], 0)),
            pl.BlockSpec((1, 2 * D_FF, D), lambda s, sr: (sr[0, s], 0, 0)),
            pl.BlockSpec((1, D, D_FF), lambda s, sr: (sr[0, s], 0, 0)),
        ],
        out_specs=pl.BlockSpec((TM, D), lambda s, sr: (sr[1, s], 0)),
    )
    return pl.pallas_call(
        _gmm_kernel,
        grid_spec=grid_spec,
        out_shape=jax.ShapeDtypeStruct((T2, D), jnp.float32),
        compiler_params=pltpu.CompilerParams(
            vmem_limit_bytes=128 * 1024 * 1024),
    )(sched, xs, w1, w2)


def kernel(hidden_states, w1, w2, score, topk):
    del topk
    pos, wab, wbb, sched = _route(score)
    pos1 = pos.reshape(TOKENS * TOPK)
    xs = _permute(hidden_states, pos1)
    y = _gmm(sched, xs, w1, w2)
    out = _combine(y, pos1[:TOKENS], pos1[TOKENS:], wab, wbb)
    return out


# speedup vs baseline: 1.4326x; 1.4326x over previous
import functools

import jax
import jax.numpy as jnp
from jax import lax
from jax.experimental import pallas as pl
from jax.experimental.pallas import tpu as pltpu
from jax.experimental.pallas import tpu_sc as plsc

NUM_EXPERTS = 8
TOKENS = 2048
D_MODEL = 1024
D_FF = 2048
TOPK = 2

TM = 256
NT = TOKEN_TILES = (TOKENS * TOPK) // TM
SSTEPS = NT + NUM_EXPERTS - 1

NC, NS = 2, 16
NW = NC * NS


def _shift_down(a, sh):
    return jnp.concatenate([jnp.zeros((sh, a.shape[1]), a.dtype), a[:-sh, :]], axis=0)


def _lane_incl_cumsum(a):
    for sh in (1, 2, 4):
        a = a + jnp.concatenate([jnp.zeros((a.shape[0], sh), a.dtype), a[:, :-sh]], axis=1)
    return a


def _route_kernel(score_ref, pos_ref, wa_ref, wb_ref, sched_ref):
    E, S = NUM_EXPERTS, SSTEPS
    s = score_ref[...].astype(jnp.float32)
    lane = lax.broadcasted_iota(jnp.int32, s.shape, 1)
    big = jnp.asarray(E, jnp.int32)
    m1 = jnp.max(s, axis=1, keepdims=True)
    i1 = jnp.min(jnp.where(s == m1, lane, big), axis=1, keepdims=True)
    oh1 = lane == i1
    s2 = jnp.where(oh1, -jnp.inf, s)
    m2 = jnp.max(s2, axis=1, keepdims=True)
    i2 = jnp.min(jnp.where(s2 == m2, lane, big), axis=1, keepdims=True)
    oh2 = lane == i2
    r = jnp.exp(m2 - m1)
    denom = 1.0 + r
    ones16 = jnp.ones((1, 16), jnp.float32)
    wa_ref[...] = (1.0 / denom) * ones16
    wb_ref[...] = (r / denom) * ones16

    A = jnp.concatenate([oh1, oh2], axis=0).astype(jnp.float32)
    inc = A
    sh = 1
    while sh < A.shape[0]:
        inc = inc + _shift_down(inc, sh)
        sh *= 2
    counts = inc[A.shape[0] - 1:, :]
    offs_incl = _lane_incl_cumsum(counts)
    offs_excl = offs_incl - counts
    ex_rank = inc - A
    pos = jnp.sum((ex_rank + offs_excl) * A, axis=1, keepdims=True)
    pos_ref[...] = pos.astype(jnp.int32)

    off = offs_excl.astype(jnp.int32)
    offn = offs_incl.astype(jnp.int32)
    cnt = counts.astype(jnp.int32)
    nz = cnt > 0
    t0 = off // TM
    t1 = jnp.where(nz, (offn - 1) // TM, 0)
    ntiles = jnp.where(nz, t1 - t0 + 1, 0)
    cumincl = _lane_incl_cumsum(ntiles)
    cumex = cumincl - ntiles
    total = cumincl[:, E - 1:]

    sub = lax.broadcasted_iota(jnp.int32, (E, E), 0)
    lan = lax.broadcasted_iota(jnp.int32, (E, E), 1)
    ident = (sub == lan).astype(jnp.float32)

    def _t(v):
        return lax.dot_general(ident, v.astype(jnp.float32), (((1,), (1,)), ((), ())),
                               preferred_element_type=jnp.float32).astype(jnp.int32)

    cuminclT, cumexT, t0T, offT, offnT = map(_t, (cumincl, cumex, t0, off, offn))
    s_iota = lax.broadcasted_iota(jnp.int32, (1, S), 1)
    e_s = jnp.sum((s_iota >= cuminclT).astype(jnp.int32), axis=0, keepdims=True)
    e_s = jnp.minimum(e_s, E - 1)
    ohE = lax.broadcasted_iota(jnp.int32, (E, S), 0) == e_s

    def _sel(vT):
        return jnp.sum(jnp.where(ohE, vT, 0), axis=0, keepdims=True)

    tile = _sel(t0T) + s_iota - _sel(cumexT)
    valid = s_iota < total
    tile = jnp.where(valid, tile, NT - 1)
    start = jnp.where(valid, jnp.maximum(_sel(offT), tile * TM), 0)
    end = jnp.where(valid, jnp.minimum(_sel(offnT), (tile + 1) * TM), 0)
    prev_tile = jnp.concatenate([jnp.full((1, 1), -1, jnp.int32), tile[:, :-1]], axis=1)
    first = (tile != prev_tile).astype(jnp.int32)
    sched_ref[...] = jnp.concatenate([e_s, tile, start, end, first], axis=0)


def _gmm_kernel(sched_ref, x_ref, w1_ref, w2_ref, y_ref):
    s = pl.program_id(0)

    @pl.when(sched_ref[4, s] == 1)
    def _():
        y_ref[...] = jnp.zeros_like(y_ref)

    x = x_ref[...]
    gate = lax.dot_general(x, w1_ref[0, :D_FF, :], (((1,), (1,)), ((), ())),
                           preferred_element_type=jnp.float32)
    up = lax.dot_general(x, w1_ref[0, D_FF:, :], (((1,), (1,)), ((), ())),
                         preferred_element_type=jnp.float32)
    h = gate * (1.0 / (1.0 + jnp.exp(-gate))) * up
    rows = sched_ref[1, s] * TM + lax.broadcasted_iota(jnp.int32, (TM, 1), 0)
    mask = (rows >= sched_ref[2, s]) & (rows < sched_ref[3, s])
    h = jnp.where(mask, h, 0.0)
    y_ref[...] += lax.dot_general(h, w2_ref[0], (((1,), (1,)), ((), ())),
                                  preferred_element_type=jnp.float32)


def _sc_mesh():
    return plsc.VectorSubcoreMesh(core_axis_name="c", subcore_axis_name="s")


def _permute(hidden, pos1):
    @functools.partial(
        pl.kernel, mesh=_sc_mesh(),
        out_type=jax.ShapeDtypeStruct((TOKENS * TOPK, D_MODEL), jnp.float32),
        scratch_types=[
            pltpu.VMEM((64,), jnp.int32),
            pltpu.VMEM((64, D_MODEL), jnp.float32),
            pltpu.SemaphoreType.DMA,
        ],
    )
    def _permute_kernel(hid_hbm, pos_hbm, xs_hbm, idx_v, rows_v, sem):
        wid = lax.axis_index("s") * NC + lax.axis_index("c")
        for c in range(2):
            jb = wid * 128 + c * 64
            tb = lax.rem(jb, TOKENS)
            pltpu.sync_copy(hid_hbm.at[pl.ds(tb, 64)], rows_v)
            pltpu.sync_copy(pos_hbm.at[pl.ds(jb, 64)], idx_v)
            pltpu.async_copy(rows_v, xs_hbm.at[idx_v], sem).wait()

    return _permute_kernel(hidden, pos1)


def _combine(y, pa, pb, wab, wbb):
    @functools.partial(
        pl.kernel, mesh=_sc_mesh(),
        out_type=jax.ShapeDtypeStruct((TOKENS, D_MODEL), jnp.float32),
        scratch_types=[
            pltpu.VMEM((32,), jnp.int32),
            pltpu.VMEM((32,), jnp.int32),
            pltpu.VMEM((32, D_MODEL), jnp.float32),
            pltpu.VMEM((32, D_MODEL), jnp.float32),
            pltpu.VMEM((32, 16), jnp.float32),
            pltpu.VMEM((32, 16), jnp.float32),
            pltpu.SemaphoreType.DMA,
        ],
    )
    def _combine_kernel(y_hbm, pa_hbm, pb_hbm, wa_hbm, wb_hbm, out_hbm,
                        idxa_v, idxb_v, a_v, b_v, wa_v, wb_v, sem):
        wid = lax.axis_index("s") * NC + lax.axis_index("c")
        for c in range(2):
            tb = wid * 64 + c * 32
            pltpu.sync_copy(pa_hbm.at[pl.ds(tb, 32)], idxa_v)
            pltpu.sync_copy(pb_hbm.at[pl.ds(tb, 32)], idxb_v)
            pltpu.sync_copy(wa_hbm.at[pl.ds(tb, 32)], wa_v)
            pltpu.sync_copy(wb_hbm.at[pl.ds(tb, 32)], wb_v)
            ca = pltpu.async_copy(y_hbm.at[idxa_v], a_v, sem)
            cb = pltpu.async_copy(y_hbm.at[idxb_v], b_v, sem)
            ca.wait()
            cb.wait()

            def row_body(rr, carry):
                wa = wa_v[rr]
                wb = wb_v[rr]
                for cc in range(D_MODEL // 16):
                    sl = pl.ds(cc * 16, 16)
                    a_v[rr, sl] = wa * a_v[rr, sl] + wb * b_v[rr, sl]
                return carry

            lax.fori_loop(0, 32, row_body, 0)
            pltpu.sync_copy(a_v, out_hbm.at[pl.ds(tb, 32)])

    return _combine_kernel(y, pa, pb, wab, wbb)


def _route(score):
    return pl.pallas_call(
        _route_kernel,
        out_shape=(
            jax.ShapeDtypeStruct((TOKENS * TOPK, 1), jnp.int32),
            jax.ShapeDtypeStruct((TOKENS, 16), jnp.float32),
            jax.ShapeDtypeStruct((TOKENS, 16), jnp.float32),
            jax.ShapeDtypeStruct((5, SSTEPS), jnp.int32),
        ),
    )(score)


def _gmm(sched, xs, w1, w2):
    T2, D = xs.shape
    grid_spec = pltpu.PrefetchScalarGridSpec(
        num_scalar_prefetch=1,
        grid=(SSTEPS,),
        in_specs=[
            pl.BlockSpec((TM, D), lambda s, sr: (sr[1, s], 0)),
            pl.BlockSpec((1, 2 * D_FF, D), lambda s, sr: (sr[0, s], 0, 0)),
            pl.BlockSpec((1, D, D_FF), lambda s, sr: (sr[0, s], 0, 0)),
        ],
        out_specs=pl.BlockSpec((TM, D), lambda s, sr: (sr[1, s], 0)),
    )
    return pl.pallas_call(
        _gmm_kernel,
        grid_spec=grid_spec,
        out_shape=jax.ShapeDtypeStruct((T2, D), jnp.float32),
        compiler_params=pltpu.CompilerParams(
            vmem_limit_bytes=128 * 1024 * 1024),
    )(sched, xs, w1, w2)


def kernel(hidden_states, w1, w2, score, topk):
    del topk
    pos, wab, wbb, sched = _route(score)
    pos1 = pos.reshape(TOKENS * TOPK)
    xs = _permute(hidden_states, pos1)
    y = _gmm(sched, xs, w1, w2)
    out = _combine(y, pos1[:TOKENS], pos1[TOKENS:], wab, wbb)
    return out


# speedup vs baseline: 1.5195x; 1.0607x over previous
import functools

import jax
import jax.numpy as jnp
from jax import lax
from jax.experimental import pallas as pl
from jax.experimental.pallas import tpu as pltpu
from jax.experimental.pallas import tpu_sc as plsc

NUM_EXPERTS = 8
TOKENS = 2048
D_MODEL = 1024
D_FF = 2048
TOPK = 2

TM = 256
NT = TOKEN_TILES = (TOKENS * TOPK) // TM
SSTEPS = NT + NUM_EXPERTS - 1
ROWS_PAD = SSTEPS * TM

NC, NS = 2, 16
NW = NC * NS


def _shift_down(a, sh):
    return jnp.concatenate([jnp.zeros((sh, a.shape[1]), a.dtype), a[:-sh, :]], axis=0)


def _lane_incl_cumsum(a):
    for sh in (1, 2, 4):
        a = a + jnp.concatenate([jnp.zeros((a.shape[0], sh), a.dtype), a[:, :-sh]], axis=1)
    return a


def _route_kernel(score_ref, pos_ref, wa_ref, wb_ref, sched_ref):
    E, S = NUM_EXPERTS, SSTEPS
    s = score_ref[...].astype(jnp.float32)
    lane = lax.broadcasted_iota(jnp.int32, s.shape, 1)
    big = jnp.asarray(E, jnp.int32)
    m1 = jnp.max(s, axis=1, keepdims=True)
    i1 = jnp.min(jnp.where(s == m1, lane, big), axis=1, keepdims=True)
    oh1 = lane == i1
    s2 = jnp.where(oh1, -jnp.inf, s)
    m2 = jnp.max(s2, axis=1, keepdims=True)
    i2 = jnp.min(jnp.where(s2 == m2, lane, big), axis=1, keepdims=True)
    oh2 = lane == i2
    r = jnp.exp(m2 - m1)
    denom = 1.0 + r
    ones16 = jnp.ones((1, 16), jnp.float32)
    wa_ref[...] = (1.0 / denom) * ones16
    wb_ref[...] = (r / denom) * ones16

    A = jnp.concatenate([oh1, oh2], axis=0).astype(jnp.float32)
    inc = A
    sh = 1
    while sh < A.shape[0]:
        inc = inc + _shift_down(inc, sh)
        sh *= 2
    counts = inc[A.shape[0] - 1:, :]
    cnt = counts.astype(jnp.int32)
    ntiles = (cnt + TM - 1) // TM
    cumincl = _lane_incl_cumsum(ntiles)
    opad = (cumincl - ntiles) * TM
    total = cumincl[:, E - 1:]
    ex_rank = inc - A
    pos = jnp.sum((ex_rank + opad.astype(jnp.float32)) * A, axis=1, keepdims=True)
    pos_ref[...] = pos.astype(jnp.int32)

    sub = lax.broadcasted_iota(jnp.int32, (E, E), 0)
    lan = lax.broadcasted_iota(jnp.int32, (E, E), 1)
    ident = (sub == lan).astype(jnp.float32)
    cuminclT = lax.dot_general(ident, cumincl.astype(jnp.float32),
                               (((1,), (1,)), ((), ())),
                               preferred_element_type=jnp.float32).astype(jnp.int32)
    s_iota = lax.broadcasted_iota(jnp.int32, (1, S), 1)
    e_s = jnp.sum((s_iota >= cuminclT).astype(jnp.int32), axis=0, keepdims=True)
    e_s = jnp.minimum(e_s, E - 1)
    valid = s_iota < total
    tile = jnp.where(valid, s_iota, S - 1)
    sched_ref[...] = jnp.concatenate([e_s, tile, valid.astype(jnp.int32)], axis=0)


def _gmm_kernel(sched_ref, x_ref, w1_ref, w2_ref, y_ref):
    s = pl.program_id(0)

    @pl.when(sched_ref[2, s] == 1)
    def _():
        x = x_ref[...]
        gate = lax.dot_general(x, w1_ref[0, :D_FF, :], (((1,), (1,)), ((), ())),
                               preferred_element_type=jnp.float32)
        up = lax.dot_general(x, w1_ref[0, D_FF:, :], (((1,), (1,)), ((), ())),
                             preferred_element_type=jnp.float32)
        h = gate * (1.0 / (1.0 + jnp.exp(-gate))) * up
        y_ref[...] = lax.dot_general(h, w2_ref[0], (((1,), (1,)), ((), ())),
                                     preferred_element_type=jnp.float32)


def _sc_mesh():
    return plsc.VectorSubcoreMesh(core_axis_name="c", subcore_axis_name="s")


def _permute(hidden, pos1):
    @functools.partial(
        pl.kernel, mesh=_sc_mesh(),
        out_type=jax.ShapeDtypeStruct((ROWS_PAD, D_MODEL), jnp.float32),
        scratch_types=[
            pltpu.VMEM((64,), jnp.int32),
            pltpu.VMEM((64, D_MODEL), jnp.float32),
            pltpu.SemaphoreType.DMA,
        ],
    )
    def _permute_kernel(hid_hbm, pos_hbm, xs_hbm, idx_v, rows_v, sem):
        wid = lax.axis_index("s") * NC + lax.axis_index("c")
        for c in range(2):
            jb = wid * 128 + c * 64
            tb = lax.rem(jb, TOKENS)
            pltpu.sync_copy(hid_hbm.at[pl.ds(tb, 64)], rows_v)
            pltpu.sync_copy(pos_hbm.at[pl.ds(jb, 64)], idx_v)
            pltpu.async_copy(rows_v, xs_hbm.at[idx_v], sem).wait()

    return _permute_kernel(hidden, pos1)


def _combine(y, pa, pb, wab, wbb):
    @functools.partial(
        pl.kernel, mesh=_sc_mesh(),
        out_type=jax.ShapeDtypeStruct((TOKENS, D_MODEL), jnp.float32),
        scratch_types=[
            pltpu.VMEM((32,), jnp.int32),
            pltpu.VMEM((32,), jnp.int32),
            pltpu.VMEM((32, D_MODEL), jnp.float32),
            pltpu.VMEM((32, D_MODEL), jnp.float32),
            pltpu.VMEM((32, 16), jnp.float32),
            pltpu.VMEM((32, 16), jnp.float32),
            pltpu.SemaphoreType.DMA,
        ],
    )
    def _combine_kernel(y_hbm, pa_hbm, pb_hbm, wa_hbm, wb_hbm, out_hbm,
                        idxa_v, idxb_v, a_v, b_v, wa_v, wb_v, sem):
        wid = lax.axis_index("s") * NC + lax.axis_index("c")
        for c in range(2):
            tb = wid * 64 + c * 32
            pltpu.sync_copy(pa_hbm.at[pl.ds(tb, 32)], idxa_v)
            pltpu.sync_copy(pb_hbm.at[pl.ds(tb, 32)], idxb_v)
            pltpu.sync_copy(wa_hbm.at[pl.ds(tb, 32)], wa_v)
            pltpu.sync_copy(wb_hbm.at[pl.ds(tb, 32)], wb_v)
            ca = pltpu.async_copy(y_hbm.at[idxa_v], a_v, sem)
            cb = pltpu.async_copy(y_hbm.at[idxb_v], b_v, sem)
            ca.wait()
            cb.wait()

            def row_body(rr, carry):
                wa = wa_v[rr]
                wb = wb_v[rr]
                for cc in range(D_MODEL // 16):
                    sl = pl.ds(cc * 16, 16)
                    a_v[rr, sl] = wa * a_v[rr, sl] + wb * b_v[rr, sl]
                return carry

            lax.fori_loop(0, 32, row_body, 0)
            pltpu.sync_copy(a_v, out_hbm.at[pl.ds(tb, 32)])

    return _combine_kernel(y, pa, pb, wab, wbb)


def _route(score):
    return pl.pallas_call(
        _route_kernel,
        out_shape=(
            jax.ShapeDtypeStruct((TOKENS * TOPK, 1), jnp.int32),
            jax.ShapeDtypeStruct((TOKENS, 16), jnp.float32),
            jax.ShapeDtypeStruct((TOKENS, 16), jnp.float32),
            jax.ShapeDtypeStruct((3, SSTEPS), jnp.int32),
        ),
    )(score)


def _gmm(sched, xs, w1, w2):
    T2, D = xs.shape
    grid_spec = pltpu.PrefetchScalarGridSpec(
        num_scalar_prefetch=1,
        grid=(SSTEPS,),
        in_specs=[
            pl.BlockSpec((TM, D), lambda s, sr: (sr[1, s], 0)),
            pl.BlockSpec((1, 2 * D_FF, D), lambda s, sr: (sr[0, s], 0, 0)),
            pl.BlockSpec((1, D, D_FF), lambda s, sr: (sr[0, s], 0, 0)),
        ],
        out_specs=pl.BlockSpec((TM, D), lambda s, sr: (sr[1, s], 0)),
    )
    return pl.pallas_call(
        _gmm_kernel,
        grid_spec=grid_spec,
        out_shape=jax.ShapeDtypeStruct((T2, D), jnp.float32),
        compiler_params=pltpu.CompilerParams(
            vmem_limit_bytes=128 * 1024 * 1024),
    )(sched, xs, w1, w2)


def kernel(hidden_states, w1, w2, score, topk):
    del topk
    pos, wab, wbb, sched = _route(score)
    pos1 = pos.reshape(TOKENS * TOPK)
    xs = _permute(hidden_states, pos1)
    y = _gmm(sched, xs, w1, w2)
    out = _combine(y, pos1[:TOKENS], pos1[TOKENS:], wab, wbb)
    return out


# speedup vs baseline: 1.5533x; 1.0223x over previous
import functools

import jax
import jax.numpy as jnp
from jax import lax
from jax.experimental import pallas as pl
from jax.experimental.pallas import tpu as pltpu
from jax.experimental.pallas import tpu_sc as plsc

NUM_EXPERTS = 8
TOKENS = 2048
D_MODEL = 1024
D_FF = 2048
TOPK = 2

TM = 256
NT = TOKEN_TILES = (TOKENS * TOPK) // TM
SSTEPS = NT + NUM_EXPERTS - 1
ROWS_PAD = SSTEPS * TM

NC, NS = 2, 16
NW = NC * NS


def _shift_down(a, sh):
    return jnp.concatenate([jnp.zeros((sh, a.shape[1]), a.dtype), a[:-sh, :]], axis=0)


def _lane_incl_cumsum(a):
    for sh in (1, 2, 4):
        a = a + jnp.concatenate([jnp.zeros((a.shape[0], sh), a.dtype), a[:, :-sh]], axis=1)
    return a


def _route_kernel(score_ref, pos_ref, wa_ref, wb_ref, sched_ref):
    E, S = NUM_EXPERTS, SSTEPS
    s = score_ref[...].astype(jnp.float32)
    lane = lax.broadcasted_iota(jnp.int32, s.shape, 1)
    big = jnp.asarray(E, jnp.int32)
    m1 = jnp.max(s, axis=1, keepdims=True)
    i1 = jnp.min(jnp.where(s == m1, lane, big), axis=1, keepdims=True)
    oh1 = lane == i1
    s2 = jnp.where(oh1, -jnp.inf, s)
    m2 = jnp.max(s2, axis=1, keepdims=True)
    i2 = jnp.min(jnp.where(s2 == m2, lane, big), axis=1, keepdims=True)
    oh2 = lane == i2
    r = jnp.exp(m2 - m1)
    denom = 1.0 + r
    ones16 = jnp.ones((1, 16), jnp.float32)
    wa_ref[...] = (1.0 / denom) * ones16
    wb_ref[...] = (r / denom) * ones16

    A = jnp.concatenate([oh1, oh2], axis=0).astype(jnp.float32)
    inc = A
    sh = 1
    while sh < A.shape[0]:
        inc = inc + _shift_down(inc, sh)
        sh *= 2
    counts = inc[A.shape[0] - 1:, :]
    cnt = counts.astype(jnp.int32)
    ntiles = (cnt + TM - 1) // TM
    cumincl = _lane_incl_cumsum(ntiles)
    opad = (cumincl - ntiles) * TM
    total = cumincl[:, E - 1:]
    ex_rank = inc - A
    pos = jnp.sum((ex_rank + opad.astype(jnp.float32)) * A, axis=1, keepdims=True)
    pos_ref[...] = pos.astype(jnp.int32)

    sub = lax.broadcasted_iota(jnp.int32, (E, E), 0)
    lan = lax.broadcasted_iota(jnp.int32, (E, E), 1)
    ident = (sub == lan).astype(jnp.float32)
    cuminclT = lax.dot_general(ident, cumincl.astype(jnp.float32),
                               (((1,), (1,)), ((), ())),
                               preferred_element_type=jnp.float32).astype(jnp.int32)
    s_iota = lax.broadcasted_iota(jnp.int32, (1, S), 1)
    e_s = jnp.sum((s_iota >= cuminclT).astype(jnp.int32), axis=0, keepdims=True)
    e_s = jnp.minimum(e_s, E - 1)
    valid = s_iota < total
    tile = jnp.where(valid, s_iota, S - 1)
    sched_ref[...] = jnp.concatenate([e_s, tile, valid.astype(jnp.int32)], axis=0)


def _gmm_kernel(sched_ref, x_ref, w1_ref, w2_ref, y_ref):
    s = pl.program_id(0)

    @pl.when(sched_ref[2, s] == 1)
    def _():
        x = x_ref[...]
        gate = lax.dot_general(x, w1_ref[0, :D_FF, :], (((1,), (1,)), ((), ())),
                               preferred_element_type=jnp.float32)
        up = lax.dot_general(x, w1_ref[0, D_FF:, :], (((1,), (1,)), ((), ())),
                             preferred_element_type=jnp.float32)
        h = gate * (1.0 / (1.0 + jnp.exp(-gate))) * up
        y_ref[...] = lax.dot_general(h, w2_ref[0], (((1,), (1,)), ((), ())),
                                     preferred_element_type=jnp.float32)


def _sc_mesh():
    return plsc.VectorSubcoreMesh(core_axis_name="c", subcore_axis_name="s")


def _permute(hidden, pos1):
    NCH, CH = 4, 32

    @functools.partial(
        pl.kernel, mesh=_sc_mesh(),
        out_type=jax.ShapeDtypeStruct((ROWS_PAD, D_MODEL), jnp.float32),
        scratch_types=[
            pltpu.VMEM((NCH * CH,), jnp.int32),
            pltpu.VMEM((CH, D_MODEL), jnp.float32),
            pltpu.VMEM((CH, D_MODEL), jnp.float32),
            pltpu.SemaphoreType.DMA,
            pltpu.SemaphoreType.DMA,
            pltpu.SemaphoreType.DMA,
            pltpu.SemaphoreType.DMA,
        ],
    )
    def _permute_kernel(hid_hbm, pos_hbm, xs_hbm, idx_v, rows0, rows1, r0, r1, w0, w1s):
        wid = lax.axis_index("s") * NC + lax.axis_index("c")
        jb0 = wid * (NCH * CH)
        bufs = (rows0, rows1)
        rsem = (r0, r1)
        wsem = (w0, w1s)
        pltpu.sync_copy(pos_hbm.at[pl.ds(jb0, NCH * CH)], idx_v)

        def _read(c):
            tb = lax.rem(jb0 + c * CH, TOKENS)
            return pltpu.async_copy(hid_hbm.at[pl.ds(tb, CH)], bufs[c % 2], rsem[c % 2])

        reads = {0: _read(0)}
        writes = {}
        for c in range(NCH):
            if c + 1 < NCH and (c + 1) not in reads:
                if c - 1 >= 0:
                    writes[c - 1].wait()
                reads[c + 1] = _read(c + 1)
            reads[c].wait()
            ivs = idx_v.at[pl.ds(c * CH, CH)]
            writes[c] = pltpu.async_copy(bufs[c % 2], xs_hbm.at[ivs], wsem[c % 2])
        writes[NCH - 2].wait()
        writes[NCH - 1].wait()

    return _permute_kernel(hidden, pos1)


def _combine(y, pa, pb, wab, wbb):
    NCH, CH = 4, 16

    @functools.partial(
        pl.kernel, mesh=_sc_mesh(),
        out_type=jax.ShapeDtypeStruct((TOKENS, D_MODEL), jnp.float32),
        scratch_types=[
            pltpu.VMEM((NCH * CH,), jnp.int32),
            pltpu.VMEM((NCH * CH,), jnp.int32),
            pltpu.VMEM((NCH * CH, 16), jnp.float32),
            pltpu.VMEM((NCH * CH, 16), jnp.float32),
            pltpu.VMEM((CH, D_MODEL), jnp.float32),
            pltpu.VMEM((CH, D_MODEL), jnp.float32),
            pltpu.VMEM((CH, D_MODEL), jnp.float32),
            pltpu.VMEM((CH, D_MODEL), jnp.float32),
            pltpu.SemaphoreType.DMA,
            pltpu.SemaphoreType.DMA,
            pltpu.SemaphoreType.DMA,
            pltpu.SemaphoreType.DMA,
        ],
    )
    def _combine_kernel(y_hbm, pa_hbm, pb_hbm, wa_hbm, wb_hbm, out_hbm,
                        idxa_v, idxb_v, wa_v, wb_v, a0, a1, b0, b1,
                        g0, g1, o0, o1):
        wid = lax.axis_index("s") * NC + lax.axis_index("c")
        tb0 = wid * (NCH * CH)
        abuf, bbuf = (a0, a1), (b0, b1)
        gsem, osem = (g0, g1), (o0, o1)
        pltpu.sync_copy(pa_hbm.at[pl.ds(tb0, NCH * CH)], idxa_v)
        pltpu.sync_copy(pb_hbm.at[pl.ds(tb0, NCH * CH)], idxb_v)
        pltpu.sync_copy(wa_hbm.at[pl.ds(tb0, NCH * CH)], wa_v)
        pltpu.sync_copy(wb_hbm.at[pl.ds(tb0, NCH * CH)], wb_v)

        def _gather(c):
            ia = idxa_v.at[pl.ds(c * CH, CH)]
            ib = idxb_v.at[pl.ds(c * CH, CH)]
            return (pltpu.async_copy(y_hbm.at[ia], abuf[c % 2], gsem[c % 2]),
                    pltpu.async_copy(y_hbm.at[ib], bbuf[c % 2], gsem[c % 2]))

        gath = {0: _gather(0)}
        outw = {}
        for c in range(NCH):
            if c + 1 < NCH:
                if c - 1 >= 0:
                    outw[c - 1].wait()
                gath[c + 1] = _gather(c + 1)
            gath[c][0].wait()
            gath[c][1].wait()
            av, bv = abuf[c % 2], bbuf[c % 2]

            def row_body(rr, carry):
                wa = wa_v[c * CH + rr]
                wb = wb_v[c * CH + rr]
                for cc in range(D_MODEL // 16):
                    sl = pl.ds(cc * 16, 16)
                    av[rr, sl] = wa * av[rr, sl] + wb * bv[rr, sl]
                return carry

            lax.fori_loop(0, CH, row_body, 0)
            outw[c] = pltpu.async_copy(
                av, out_hbm.at[pl.ds(tb0 + c * CH, CH)], osem[c % 2])
        outw[NCH - 2].wait()
        outw[NCH - 1].wait()

    return _combine_kernel(y, pa, pb, wab, wbb)


def _route(score):
    return pl.pallas_call(
        _route_kernel,
        out_shape=(
            jax.ShapeDtypeStruct((TOKENS * TOPK, 1), jnp.int32),
            jax.ShapeDtypeStruct((TOKENS, 16), jnp.float32),
            jax.ShapeDtypeStruct((TOKENS, 16), jnp.float32),
            jax.ShapeDtypeStruct((3, SSTEPS), jnp.int32),
        ),
    )(score)


def _gmm(sched, xs, w1, w2):
    T2, D = xs.shape
    grid_spec = pltpu.PrefetchScalarGridSpec(
        num_scalar_prefetch=1,
        grid=(SSTEPS,),
        in_specs=[
            pl.BlockSpec((TM, D), lambda s, sr: (sr[1, s], 0)),
            pl.BlockSpec((1, 2 * D_FF, D), lambda s, sr: (sr[0, s], 0, 0)),
            pl.BlockSpec((1, D, D_FF), lambda s, sr: (sr[0, s], 0, 0)),
        ],
        out_specs=pl.BlockSpec((TM, D), lambda s, sr: (sr[1, s], 0)),
    )
    return pl.pallas_call(
        _gmm_kernel,
        grid_spec=grid_spec,
        out_shape=jax.ShapeDtypeStruct((T2, D), jnp.float32),
        compiler_params=pltpu.CompilerParams(
            vmem_limit_bytes=128 * 1024 * 1024),
    )(sched, xs, w1, w2)


def kernel(hidden_states, w1, w2, score, topk):
    del topk
    pos, wab, wbb, sched = _route(score)
    pos1 = pos.reshape(TOKENS * TOPK)
    xs = _permute(hidden_states, pos1)
    y = _gmm(sched, xs, w1, w2)
    out = _combine(y, pos1[:TOKENS], pos1[TOKENS:], wab, wbb)
    return out
